# Initial kernel scaffold; baseline (speedup 1.0000x reference)
import jax
import jax.numpy as jnp
from jax.experimental import pallas as pl


def kernel(p, x, o, f, Wq, bq, Wk, bk, Wv, bv, Wp1, bp1, g_p, b_p, Wp2, bp2, g_a, b_a, Ww1, bw1, g_b, b_b, Ww2, bw2):
    raise NotImplementedError("write your pallas kernel here")


# speedup vs baseline: 3.5636x; 3.5636x over previous
import functools

import jax
import jax.numpy as jnp
from jax import lax
from jax.experimental import pallas as pl
from jax.experimental.pallas import tpu as pltpu
from jax.experimental.pallas import tpu_sc as plsc

N = 8192
CIN = 128
MID = 128
OUTP = 128
NSAMP = 16
SHARE = 8
WCH = OUTP // SHARE
ROWS = N * NSAMP
TW = 384
EPS = 1e-5
BM = 512
BR = BM * NSAMP
CNT = float(ROWS)


def _proj_body(x_ref, pp_ref, w_ref, b_ref, q_ref, t_ref):
    y = jnp.dot(x_ref[...], w_ref[...], preferred_element_type=jnp.float32)
    y = y + b_ref[...]
    q_ref[...] = y[:, :MID]
    bm = y.shape[0]
    t_ref[...] = jnp.concatenate(
        [y[:, MID:], pp_ref[...], jnp.zeros((bm, TW - 2 * MID - 16), jnp.float32)],
        axis=1)


def _proj(x, pp16, w, b):
    bm = 1024
    return pl.pallas_call(
        _proj_body,
        grid=(N // bm,),
        in_specs=[
            pl.BlockSpec((bm, CIN), lambda i: (i, 0)),
            pl.BlockSpec((bm, 16), lambda i: (i, 0)),
            pl.BlockSpec((CIN, 3 * MID), lambda i: (0, 0)),
            pl.BlockSpec((1, 3 * MID), lambda i: (0, 0)),
        ],
        out_specs=[
            pl.BlockSpec((bm, MID), lambda i: (i, 0)),
            pl.BlockSpec((bm, TW), lambda i: (i, 0)),
        ],
        out_shape=[
            jax.ShapeDtypeStruct((N, MID), jnp.float32),
            jax.ShapeDtypeStruct((N, TW), jnp.float32),
        ],
    )(x, pp16, w, b)


def _knn_body(pb_ref, pt_ref, o_ref):
    pb = pb_ref[...]
    pt = pt_ref[...]
    sq_r = jnp.sum(pb * pb, axis=1, keepdims=True)
    sq_c = jnp.sum(pt * pt, axis=0, keepdims=True)
    d2 = sq_r + sq_c - 2.0 * jnp.dot(pb, pt, preferred_element_type=jnp.float32)
    iota = lax.broadcasted_iota(jnp.int32, (BM, N), 1)
    kcol = lax.broadcasted_iota(jnp.int32, (BM, NSAMP), 1)

    def step(k, carry):
        d2c, acc = carry
        m = jnp.min(d2c, axis=1, keepdims=True)
        amin = jnp.min(jnp.where(d2c == m, iota, N), axis=1, keepdims=True)
        acc = jnp.where(kcol == k, amin, acc)
        d2c = jnp.where(iota == amin, jnp.inf, d2c)
        return d2c, acc

    _, acc = lax.fori_loop(0, NSAMP, step, (d2, jnp.zeros((BM, NSAMP), jnp.int32)))
    o_ref[...] = acc


def _knn(pp8, pt8):
    return pl.pallas_call(
        _knn_body,
        grid=(N // BM,),
        in_specs=[
            pl.BlockSpec((BM, 8), lambda i: (i, 0)),
            pl.BlockSpec((8, N), lambda i: (0, 0)),
        ],
        out_specs=pl.BlockSpec((BM, NSAMP), lambda i: (i, 0)),
        out_shape=jax.ShapeDtypeStruct((N, NSAMP), jnp.int32),
    )(pp8, pt8)


def _sc_gather(table, idxf):
    nw = 32
    per_w = ROWS // nw
    ch = 128
    nch = per_w // ch
    mesh = plsc.VectorSubcoreMesh(core_axis_name="c", subcore_axis_name="s")

    @functools.partial(
        pl.kernel,
        mesh=mesh,
        out_type=jax.ShapeDtypeStruct((ROWS, TW), jnp.float32),
        scratch_types=[
            pltpu.VMEM((ch,), jnp.int32),
            pltpu.VMEM((ch, TW), jnp.float32),
            pltpu.SemaphoreType.DMA,
        ],
    )
    def k(tab_hbm, idx_hbm, out_hbm, idx_v, rows_v, sem):
        wid = lax.axis_index("s") * 2 + lax.axis_index("c")

        def chunk(i, carry):
            base = wid * per_w + i * ch
            pltpu.sync_copy(idx_hbm.at[pl.ds(base, ch)], idx_v)
            pltpu.async_copy(tab_hbm.at[idx_v], rows_v, sem).wait()
            pltpu.sync_copy(rows_v, out_hbm.at[pl.ds(base, ch)])
            return carry

        lax.fori_loop(0, nch, chunk, 0)

    return k(table, idxf)


def _expand_rows(a, width):
    return jnp.broadcast_to(a[:, None, :], (BM, NSAMP, width)).reshape(BR, width)


def _bn_relu(t, ssum, ssq, g, b):
    m = ssum / CNT
    v = ssq / CNT - m * m
    s = g * lax.rsqrt(v + EPS)
    return jnp.maximum(t * s + (b - m * s), 0.0)


def _compute_pr(t, tsum, tsq, gp, bp, wp2, bp2):
    a = _bn_relu(t, tsum, tsq, gp, bp)
    return jnp.dot(a, wp2, preferred_element_type=jnp.float32) + bp2


def _stat_init(i, refs):
    @pl.when(i == 0)
    def _():
        for r in refs:
            r[...] = jnp.zeros_like(r)


_FULL = lambda shape: pl.BlockSpec(shape, lambda i: tuple(0 for _ in shape))


def _tstats_body(pg_ref, pc_ref, w_ref, b_ref, t_ref, osum_ref, osq_ref):
    p_r = pg_ref[...][:, :16] - _expand_rows(pc_ref[...], 16)
    t = b_ref[...] + (p_r[:, 0:1] * w_ref[0:1, :]
                      + p_r[:, 1:2] * w_ref[1:2, :]
                      + p_r[:, 2:3] * w_ref[2:3, :])
    t_ref[...] = t
    _stat_init(pl.program_id(0), (osum_ref, osq_ref))
    osum_ref[...] += jnp.sum(t, axis=0, keepdims=True)
    osq_ref[...] += jnp.sum(t * t, axis=0, keepdims=True)


def _tstats(tab_g, pp16, wp1, bp1):
    return pl.pallas_call(
        _tstats_body,
        grid=(N // BM,),
        in_specs=[
            pl.BlockSpec((BR, MID), lambda i: (i, 2)),
            pl.BlockSpec((BM, 16), lambda i: (i, 0)),
            _FULL((16, 16)),
            _FULL((1, 16)),
        ],
        out_specs=[
            pl.BlockSpec((BR, 16), lambda i: (i, 0)),
            _FULL((1, 16)), _FULL((1, 16)),
        ],
        out_shape=[
            jax.ShapeDtypeStruct((ROWS, 16), jnp.float32),
            jax.ShapeDtypeStruct((1, 16), jnp.float32),
            jax.ShapeDtypeStruct((1, 16), jnp.float32),
        ],
    )(tab_g, pp16, wp1, bp1)


def _rstats_body(xkg_ref, t_ref, xq_ref,
                 tsum_ref, tsq_ref, gp_ref, bp_ref, wp2_ref, bp2_ref,
                 osum_ref, osq_ref):
    pr = _compute_pr(t_ref[...], tsum_ref[...], tsq_ref[...],
                     gp_ref[...], bp_ref[...], wp2_ref[...], bp2_ref[...])
    r = xkg_ref[...] - _expand_rows(xq_ref[...], MID) + pr
    _stat_init(pl.program_id(0), (osum_ref, osq_ref))
    osum_ref[...] += jnp.sum(r, axis=0, keepdims=True)
    osq_ref[...] += jnp.sum(r * r, axis=0, keepdims=True)


def _rstats(tab_g, t16, xq, tsum, tsq, gp, bp, wp2, bp2):
    return pl.pallas_call(
        _rstats_body,
        grid=(N // BM,),
        in_specs=[
            pl.BlockSpec((BR, MID), lambda i: (i, 0)),
            pl.BlockSpec((BR, 16), lambda i: (i, 0)),
            pl.BlockSpec((BM, MID), lambda i: (i, 0)),
            _FULL((1, 16)), _FULL((1, 16)), _FULL((1, 16)), _FULL((1, 16)),
            _FULL((16, MID)), _FULL((1, MID)),
        ],
        out_specs=[_FULL((1, MID)), _FULL((1, MID))],
        out_shape=[jax.ShapeDtypeStruct((1, MID), jnp.float32)] * 2,
    )(tab_g, t16, xq, tsum, tsq, gp, bp, wp2, bp2)


def _wpass_body(xkg_ref, t_ref, xq_ref,
                tsum_ref, tsq_ref, gp_ref, bp_ref, wp2_ref, bp2_ref,
                rsum_ref, rsq_ref, ga_ref, ba_ref, ww1_ref, bw1_ref,
                w1_ref, osum_ref, osq_ref):
    pr = _compute_pr(t_ref[...], tsum_ref[...], tsq_ref[...],
                     gp_ref[...], bp_ref[...], wp2_ref[...], bp2_ref[...])
    r = xkg_ref[...] - _expand_rows(xq_ref[...], MID) + pr
    a = _bn_relu(r, rsum_ref[...], rsq_ref[...], ga_ref[...], ba_ref[...])
    w1 = jnp.dot(a, ww1_ref[...], preferred_element_type=jnp.float32) + bw1_ref[...]
    w1_ref[...] = w1
    _stat_init(pl.program_id(0), (osum_ref, osq_ref))
    osum_ref[...] += jnp.sum(w1, axis=0, keepdims=True)
    osq_ref[...] += jnp.sum(w1 * w1, axis=0, keepdims=True)


def _wpass(tab_g, t16, xq, tsum, tsq, gp, bp, wp2, bp2,
           rsum, rsq, ga, ba, ww1, bw1):
    return pl.pallas_call(
        _wpass_body,
        grid=(N // BM,),
        in_specs=[
            pl.BlockSpec((BR, MID), lambda i: (i, 0)),
            pl.BlockSpec((BR, 16), lambda i: (i, 0)),
            pl.BlockSpec((BM, MID), lambda i: (i, 0)),
            _FULL((1, 16)), _FULL((1, 16)), _FULL((1, 16)), _FULL((1, 16)),
            _FULL((16, MID)), _FULL((1, MID)),
            _FULL((1, MID)), _FULL((1, MID)), _FULL((1, MID)), _FULL((1, MID)),
            _FULL((MID, WCH)), _FULL((1, WCH)),
        ],
        out_specs=[
            pl.BlockSpec((BR, WCH), lambda i: (i, 0)),
            _FULL((1, WCH)), _FULL((1, WCH)),
        ],
        out_shape=[
            jax.ShapeDtypeStruct((ROWS, WCH), jnp.float32),
            jax.ShapeDtypeStruct((1, WCH), jnp.float32),
            jax.ShapeDtypeStruct((1, WCH), jnp.float32),
        ],
    )(tab_g, t16, xq, tsum, tsq, gp, bp, wp2, bp2,
      rsum, rsq, ga, ba, ww1, bw1)


def _final_body(w1_ref, xvg_ref, t_ref,
                tsum_ref, tsq_ref, gp_ref, bp_ref, wp2_ref, bp2_ref,
                wsum_ref, wsq_ref, gb_ref, bb_ref, ww2_ref, bw2_ref, o_ref):
    pr = _compute_pr(t_ref[...], tsum_ref[...], tsq_ref[...],
                     gp_ref[...], bp_ref[...], wp2_ref[...], bp2_ref[...])
    v = xvg_ref[...] + pr
    h = _bn_relu(w1_ref[...], wsum_ref[...], wsq_ref[...],
                 gb_ref[...], bb_ref[...])
    logits = jnp.dot(h, ww2_ref[...], preferred_element_type=jnp.float32)
    logits = logits + bw2_ref[...]
    l3 = logits.reshape(BM, NSAMP, WCH)
    mx = jnp.max(l3, axis=1, keepdims=True)
    e3 = jnp.exp(l3 - mx)
    den = jnp.sum(e3, axis=1, keepdims=True)
    w2 = (e3 / den).reshape(BR, WCH)
    wt = jnp.concatenate([w2] * SHARE, axis=1)
    vw = (v * wt).reshape(BM, NSAMP, OUTP)
    o_ref[...] = jnp.sum(vw, axis=1)


def _final(w1, tab_g, t16, tsum, tsq, gp, bp, wp2, bp2,
           wsum, wsq, gb, bb, ww2, bw2):
    return pl.pallas_call(
        _final_body,
        grid=(N // BM,),
        in_specs=[
            pl.BlockSpec((BR, WCH), lambda i: (i, 0)),
            pl.BlockSpec((BR, MID), lambda i: (i, 1)),
            pl.BlockSpec((BR, 16), lambda i: (i, 0)),
            _FULL((1, 16)), _FULL((1, 16)), _FULL((1, 16)), _FULL((1, 16)),
            _FULL((16, MID)), _FULL((1, MID)),
            _FULL((1, WCH)), _FULL((1, WCH)), _FULL((1, WCH)), _FULL((1, WCH)),
            _FULL((WCH, WCH)), _FULL((1, WCH)),
        ],
        out_specs=pl.BlockSpec((BM, OUTP), lambda i: (i, 0)),
        out_shape=jax.ShapeDtypeStruct((N, OUTP), jnp.float32),
    )(w1, tab_g, t16, tsum, tsq, gp, bp, wp2, bp2,
      wsum, wsq, gb, bb, ww2, bw2)


def kernel(p, x, o, f, Wq, bq, Wk, bk, Wv, bv, Wp1, bp1, g_p, b_p, Wp2, bp2,
           g_a, b_a, Ww1, bw1, g_b, b_b, Ww2, bw2):
    del o, f
    pp8 = jnp.pad(p, ((0, 0), (0, 5)))
    pp16 = jnp.pad(p, ((0, 0), (0, 13)))
    pt8 = pp8.T
    w_qkv = jnp.concatenate([Wq, Wk, Wv], axis=1)
    b_qkv = jnp.concatenate([bq, bk, bv])[None, :]
    wp1 = jnp.zeros((16, 16), jnp.float32).at[:3, :3].set(Wp1)
    bp1p = jnp.pad(bp1, (0, 13))[None, :]
    gpp = jnp.pad(g_p, (0, 13))[None, :]
    bpp = jnp.pad(b_p, (0, 13))[None, :]
    wp2 = jnp.zeros((16, MID), jnp.float32).at[:3, :].set(Wp2)
    bp2p = bp2[None, :]
    gap = g_a[None, :]
    bap = b_a[None, :]
    bw1p = bw1[None, :]
    gbp = g_b[None, :]
    bbp = b_b[None, :]
    bw2p = bw2[None, :]

    xq, table = _proj(x, pp16, w_qkv, b_qkv)
    idx = _knn(pp8, pt8)
    tab_g = _sc_gather(table, idx.reshape(ROWS))
    t16, tsum, tsq = _tstats(tab_g, pp16, wp1, bp1p)
    rsum, rsq = _rstats(tab_g, t16, xq, tsum, tsq, gpp, bpp, wp2, bp2p)
    w1, wsum, wsq = _wpass(tab_g, t16, xq, tsum, tsq, gpp, bpp, wp2, bp2p,
                           rsum, rsq, gap, bap, Ww1, bw1p)
    out = _final(w1, tab_g, t16, tsum, tsq, gpp, bpp, wp2, bp2p,
                 wsum, wsq, gbp, bbp, Ww2, bw2p)
    return out
